# baseline (speedup 1.0000x reference)
import functools

import jax
import jax.numpy as jnp
from jax import lax
from jax.experimental import pallas as pl
from jax.experimental.pallas import tpu as pltpu
from jax.experimental.pallas import tpu_sc as plsc


def _tree_sum(terms):
    while len(terms) > 1:
        nxt = [terms[i] + terms[i + 1] for i in range(0, len(terms) - 1, 2)]
        if len(terms) % 2:
            nxt.append(terms[-1])
        terms = nxt
    return terms[0]


def _conv_body(x_ref, s_ref, w_ref, b_ref, o_ref):
    i = pl.program_id(0)
    e = s_ref[i]
    x = x_ref[0].astype(jnp.bfloat16)
    C, H, W = x.shape
    zc = jnp.zeros((C, H, 1), jnp.bfloat16)
    zr = jnp.zeros((1, W), jnp.bfloat16)
    xl = jnp.concatenate([zc, x[:, :, : W - 1]], axis=2)
    xr = jnp.concatenate([x[:, :, 1:], zc], axis=2)
    xs = (xl, x, xr)
    for co in range(3):
        h = []
        for ky in range(3):
            terms = [
                xs[kx][ci] * w_ref[e, co, ci, ky, kx].astype(jnp.bfloat16)
                for ci in range(3)
                for kx in range(3)
            ]
            h.append(_tree_sum(terms))
        top = jnp.concatenate([zr, h[0][: H - 1, :]], axis=0)
        bot = jnp.concatenate([h[2][1:, :], zr], axis=0)
        out = (top + h[1]) + (bot + b_ref[e, co].astype(jnp.bfloat16))
        o_ref[0, co] = out.astype(jnp.float32)


def _masks_sc_call(s):
    mesh = plsc.VectorSubcoreMesh(core_axis_name="c", subcore_axis_name="s")

    @functools.partial(
        pl.kernel,
        mesh=mesh,
        out_type=jax.ShapeDtypeStruct((48,), jnp.int32),
        scratch_types=[
            pltpu.VMEM((16,), jnp.int32),
            pltpu.VMEM((16,), jnp.int32),
        ],
    )
    def _masks_body(s_hbm, out_hbm, s_v, m_v):
        wid = lax.axis_index("s") * 2 + lax.axis_index("c")

        @pl.when(wid == 0)
        def _():
            pltpu.sync_copy(s_hbm, s_v)
            for k in range(3):
                m_v[...] = jnp.where(s_v[...] == k, jnp.int32(1), jnp.int32(0))
                pltpu.sync_copy(m_v, out_hbm.at[pl.ds(16 * k, 16)])

    return _masks_body(s)


def kernel(x, intensity, W_low, b_low, W_medium, b_medium, W_high, b_high):
    N, C, H, W = x.shape
    w_all = jnp.stack([W_low, W_medium, W_high])
    b_all = jnp.stack([b_low, b_medium, b_high])
    s = intensity.astype(jnp.int32)
    out = pl.pallas_call(
        _conv_body,
        grid=(N,),
        in_specs=[
            pl.BlockSpec((1, C, H, W), lambda i: (i, 0, 0, 0)),
            pl.BlockSpec(memory_space=pltpu.SMEM),
            pl.BlockSpec(memory_space=pltpu.SMEM),
            pl.BlockSpec(memory_space=pltpu.SMEM),
        ],
        out_specs=pl.BlockSpec((1, C, H, W), lambda i: (i, 0, 0, 0)),
        out_shape=jax.ShapeDtypeStruct((N, C, H, W), jnp.float32),
        compiler_params=pltpu.CompilerParams(
            dimension_semantics=("parallel",),
        ),
    )(x, s, w_all, b_all)
    m = _masks_sc_call(s).reshape(3, 16)
    return (out, intensity, m[0] != 0, m[1] != 0, m[2] != 0)


# speedup vs baseline: 1.2449x; 1.2449x over previous
import jax
import jax.numpy as jnp
from jax.experimental import pallas as pl
from jax.experimental.pallas import tpu as pltpu


def _tree_sum(terms):
    while len(terms) > 1:
        nxt = [terms[i] + terms[i + 1] for i in range(0, len(terms) - 1, 2)]
        if len(terms) % 2:
            nxt.append(terms[-1])
        terms = nxt
    return terms[0]


def _conv_body(x_ref, s_ref, w_ref, b_ref, o_ref):
    i = pl.program_id(0)
    e = s_ref[i]
    x = x_ref[0].astype(jnp.bfloat16)
    C, H, W = x.shape
    zc = jnp.zeros((C, H, 1), jnp.bfloat16)
    zr = jnp.zeros((1, W), jnp.bfloat16)
    xl = jnp.concatenate([zc, x[:, :, : W - 1]], axis=2)
    xr = jnp.concatenate([x[:, :, 1:], zc], axis=2)
    xs = (xl, x, xr)
    for co in range(3):
        h = []
        for ky in range(3):
            terms = [
                xs[kx][ci] * w_ref[e, co, ci, ky, kx].astype(jnp.bfloat16)
                for ci in range(3)
                for kx in range(3)
            ]
            h.append(_tree_sum(terms))
        top = jnp.concatenate([zr, h[0][: H - 1, :]], axis=0)
        bot = jnp.concatenate([h[2][1:, :], zr], axis=0)
        out = (top + h[1]) + (bot + b_ref[e, co].astype(jnp.bfloat16))
        o_ref[0, co] = out.astype(jnp.float32)


def kernel(x, intensity, W_low, b_low, W_medium, b_medium, W_high, b_high):
    N, C, H, W = x.shape
    w_all = jnp.stack([W_low, W_medium, W_high])
    b_all = jnp.stack([b_low, b_medium, b_high])
    s = intensity.astype(jnp.int32)
    out = pl.pallas_call(
        _conv_body,
        grid=(N,),
        in_specs=[
            pl.BlockSpec((1, C, H, W), lambda i: (i, 0, 0, 0)),
            pl.BlockSpec(memory_space=pltpu.SMEM),
            pl.BlockSpec(memory_space=pltpu.SMEM),
            pl.BlockSpec(memory_space=pltpu.SMEM),
        ],
        out_specs=pl.BlockSpec((1, C, H, W), lambda i: (i, 0, 0, 0)),
        out_shape=jax.ShapeDtypeStruct((N, C, H, W), jnp.float32),
        compiler_params=pltpu.CompilerParams(
            dimension_semantics=("parallel",),
        ),
    )(x, s, w_all, b_all)
    return (out, intensity, intensity == 0, intensity == 1, intensity == 2)
